# baseline (speedup 1.0000x reference)
import functools

import jax
import jax.numpy as jnp
from jax import lax
from jax.experimental import pallas as pl
from jax.experimental.pallas import tpu as pltpu
from jax.experimental.pallas import tpu_sc as plsc

_B, _S, _D = 4, 4096, 1024
_NC, _NS = 2, 16
_NW = _NC * _NS
_SW = _S // _NW
_R = 8
_C = _SW // _R
_CH = _R * _D
_NV = _CH // 16


@functools.partial(
    pl.kernel,
    out_type=jax.ShapeDtypeStruct((_B * _S * _D,), jnp.float32),
    mesh=plsc.VectorSubcoreMesh(core_axis_name="c", subcore_axis_name="s"),
    scratch_types=[
        pltpu.VMEM((2, _B, _CH), jnp.float32),
        pltpu.VMEM((2, _CH), jnp.float32),
        pltpu.SemaphoreType.DMA,
        pltpu.SemaphoreType.DMA,
        pltpu.SemaphoreType.DMA,
        pltpu.SemaphoreType.DMA,
    ],
)
def _sc_add(x_hbm, t_hbm, o_hbm, x_buf, t_buf, sem_i0, sem_i1, sem_o0, sem_o1):
    sems_in = (sem_i0, sem_i1)
    sems_out = (sem_o0, sem_o1)
    wid = lax.axis_index("s") * _NC + lax.axis_index("c")
    base = wid * _SW * _D

    copies_in = [None, None]
    copies_out = [None, None]

    def start_in(c):
        p = c % 2
        off = base + c * _CH
        lst = [pltpu.async_copy(t_hbm.at[pl.ds(off, _CH)], t_buf.at[p], sems_in[p])]
        for b in range(_B):
            lst.append(pltpu.async_copy(
                x_hbm.at[pl.ds(b * _S * _D + off, _CH)], x_buf.at[p, b], sems_in[p]))
        copies_in[p] = lst

    def start_out(c):
        p = c % 2
        off = base + c * _CH
        copies_out[p] = [
            pltpu.async_copy(x_buf.at[p, b], o_hbm.at[pl.ds(b * _S * _D + off, _CH)],
                             sems_out[p])
            for b in range(_B)
        ]

    def compute(c):
        p = c % 2

        def body(v, carry):
            o16 = v * 16
            t = t_buf[p, pl.ds(o16, 16)]
            for b in range(_B):
                x_buf[p, b, pl.ds(o16, 16)] = x_buf[p, b, pl.ds(o16, 16)] + t
            return carry

        lax.fori_loop(0, _NV, body, 0, unroll=2)

    start_in(0)
    for c in range(_C):
        p = c % 2
        if c + 1 < _C:
            if c >= 1:
                for d in copies_out[(c - 1) % 2]:
                    d.wait()
            start_in(c + 1)
        for d in copies_in[p]:
            d.wait()
        compute(c)
        start_out(c)
    for d in copies_out[_C % 2]:
        d.wait()
    for d in copies_out[(_C - 1) % 2]:
        d.wait()


def kernel(x, table):
    out = _sc_add(x.reshape(-1), table.reshape(-1))
    return out.reshape(x.shape)


# speedup vs baseline: 3.2552x; 3.2552x over previous
import functools

import jax
import jax.numpy as jnp
from jax import lax
from jax.experimental import pallas as pl
from jax.experimental.pallas import tpu as pltpu
from jax.experimental.pallas import tpu_sc as plsc

_B, _S, _D = 4, 4096, 1024
_NC, _NS = 2, 16
_NW = _NC * _NS
_SW = _S // _NW
_R = 8
_C = _SW // _R
_NV = _R * _D // 16


@functools.partial(
    pl.kernel,
    out_type=jax.ShapeDtypeStruct((_B, _S, _D), jnp.float32),
    mesh=plsc.VectorSubcoreMesh(core_axis_name="c", subcore_axis_name="s"),
    scratch_types=[
        pltpu.VMEM((2, _B, _R, _D), jnp.float32),
        pltpu.VMEM((2, _R, _D), jnp.float32),
        pltpu.SemaphoreType.DMA,
        pltpu.SemaphoreType.DMA,
        pltpu.SemaphoreType.DMA,
        pltpu.SemaphoreType.DMA,
    ],
    compiler_params=pltpu.CompilerParams(use_tc_tiling_on_sc=True),
)
def _sc_add(x_hbm, t_hbm, o_hbm, x_buf, t_buf, sem_i0, sem_i1, sem_o0, sem_o1):
    sems_in = (sem_i0, sem_i1)
    sems_out = (sem_o0, sem_o1)
    wid = lax.axis_index("s") * _NC + lax.axis_index("c")
    base_s = wid * _SW

    copies_in = [None, None]
    copies_out = [None, None]

    def start_in(c):
        p = c % 2
        s0 = base_s + c * _R
        lst = [pltpu.async_copy(t_hbm.at[pl.ds(s0, _R), :], t_buf.at[p], sems_in[p])]
        for b in range(_B):
            lst.append(pltpu.async_copy(
                x_hbm.at[b, pl.ds(s0, _R), :], x_buf.at[p, b], sems_in[p]))
        copies_in[p] = lst

    def start_out(c):
        p = c % 2
        s0 = base_s + c * _R
        copies_out[p] = [
            pltpu.async_copy(x_buf.at[p, b], o_hbm.at[b, pl.ds(s0, _R), :],
                             sems_out[p])
            for b in range(_B)
        ]

    def compute(c):
        p = c % 2

        def body(v, carry):
            r = v >> 6
            o16 = (v & 63) * 16
            t = t_buf[p, r, pl.ds(o16, 16)]
            for b in range(_B):
                x_buf[p, b, r, pl.ds(o16, 16)] = x_buf[p, b, r, pl.ds(o16, 16)] + t
            return carry

        lax.fori_loop(0, _NV, body, 0, unroll=4)

    start_in(0)
    for c in range(_C):
        p = c % 2
        if c + 1 < _C:
            if c >= 1:
                for d in copies_out[(c - 1) % 2]:
                    d.wait()
            start_in(c + 1)
        for d in copies_in[p]:
            d.wait()
        compute(c)
        start_out(c)
    for d in copies_out[_C % 2]:
        d.wait()
    for d in copies_out[(_C - 1) % 2]:
        d.wait()


def kernel(x, table):
    return _sc_add(x, table)


# speedup vs baseline: 4.1441x; 1.2731x over previous
import functools

import jax
import jax.numpy as jnp
from jax import lax
from jax.experimental import pallas as pl
from jax.experimental.pallas import tpu as pltpu
from jax.experimental.pallas import tpu_sc as plsc

_B, _S, _D = 4, 4096, 1024
_NC, _NS = 2, 16
_NW = _NC * _NS
_SW = _S // _NW
_R = 8
_C = _SW // _R
_NV = _R * _D // 16


@functools.partial(
    pl.kernel,
    out_type=jax.ShapeDtypeStruct((_B, _S, _D), jnp.float32),
    mesh=plsc.VectorSubcoreMesh(core_axis_name="c", subcore_axis_name="s"),
    scratch_types=[
        pltpu.VMEM((2, _B, _R, _D), jnp.float32),
        pltpu.VMEM((2, _R, _D), jnp.float32),
        pltpu.SemaphoreType.DMA,
        pltpu.SemaphoreType.DMA,
        pltpu.SemaphoreType.DMA,
        pltpu.SemaphoreType.DMA,
    ],
    compiler_params=pltpu.CompilerParams(use_tc_tiling_on_sc=True),
)
def _sc_add(x_hbm, t_hbm, o_hbm, x_buf, t_buf, sem_i0, sem_i1, sem_o0, sem_o1):
    sems_in = (sem_i0, sem_i1)
    sems_out = (sem_o0, sem_o1)
    wid = lax.axis_index("s") * _NC + lax.axis_index("c")
    base_s = wid * _SW

    copies_in = [None, None]
    copies_out = [None, None]

    def start_in(c):
        p = c % 2
        s0 = base_s + c * _R
        lst = [pltpu.async_copy(t_hbm.at[pl.ds(s0, _R), :], t_buf.at[p], sems_in[p])]
        for b in range(_B):
            lst.append(pltpu.async_copy(
                x_hbm.at[b, pl.ds(s0, _R), :], x_buf.at[p, b], sems_in[p]))
        copies_in[p] = lst

    def start_out(c):
        p = c % 2
        s0 = base_s + c * _R
        copies_out[p] = [
            pltpu.async_copy(x_buf.at[p, b], o_hbm.at[b, pl.ds(s0, _R), :],
                             sems_out[p])
            for b in range(_B)
        ]

    def compute(c):
        p = c % 2

        def row(r, carry):
            def body(j, carry2):
                o16 = j * 16
                t = t_buf[p, r, pl.ds(o16, 16)]
                for b in range(_B):
                    x_buf[p, b, r, pl.ds(o16, 16)] = (
                        x_buf[p, b, r, pl.ds(o16, 16)] + t)
                return carry2

            return lax.fori_loop(0, _D // 16, body, carry, unroll=8)

        lax.fori_loop(0, _R, row, 0)

    start_in(0)
    for c in range(_C):
        p = c % 2
        if c + 1 < _C:
            if c >= 1:
                for d in copies_out[(c - 1) % 2]:
                    d.wait()
            start_in(c + 1)
        for d in copies_in[p]:
            d.wait()
        compute(c)
        start_out(c)
    for d in copies_out[_C % 2]:
        d.wait()
    for d in copies_out[(_C - 1) % 2]:
        d.wait()


def kernel(x, table):
    return _sc_add(x, table)


# speedup vs baseline: 4.2224x; 1.0189x over previous
import functools

import jax
import jax.numpy as jnp
from jax import lax
from jax.experimental import pallas as pl
from jax.experimental.pallas import tpu as pltpu
from jax.experimental.pallas import tpu_sc as plsc

_B, _S, _D = 4, 4096, 1024
_NC, _NS = 2, 16
_NW = _NC * _NS
_SW = _S // _NW
_R = 8
_C = _SW // _R
_NV = _R * _D // 16


@functools.partial(
    pl.kernel,
    out_type=jax.ShapeDtypeStruct((_B, _S, _D), jnp.float32),
    mesh=plsc.VectorSubcoreMesh(core_axis_name="c", subcore_axis_name="s"),
    scratch_types=[
        pltpu.VMEM((2, _B, _R, _D), jnp.float32),
        pltpu.VMEM((2, _R, _D), jnp.float32),
        pltpu.SemaphoreType.DMA,
        pltpu.SemaphoreType.DMA,
        pltpu.SemaphoreType.DMA,
        pltpu.SemaphoreType.DMA,
    ],
    compiler_params=pltpu.CompilerParams(use_tc_tiling_on_sc=True),
)
def _sc_add(x_hbm, t_hbm, o_hbm, x_buf, t_buf, sem_i0, sem_i1, sem_o0, sem_o1):
    sems_in = (sem_i0, sem_i1)
    sems_out = (sem_o0, sem_o1)
    wid = lax.axis_index("s") * _NC + lax.axis_index("c")
    base_s = wid * _SW

    copies_in = [None, None]
    copies_out = [None, None]

    def start_in(c):
        p = c % 2
        s0 = base_s + c * _R
        copies_in[p] = [
            pltpu.async_copy(t_hbm.at[pl.ds(s0, _R), :], t_buf.at[p], sems_in[p]),
            pltpu.async_copy(x_hbm.at[:, pl.ds(s0, _R), :], x_buf.at[p],
                             sems_in[p]),
        ]

    def start_out(c):
        p = c % 2
        s0 = base_s + c * _R
        copies_out[p] = [
            pltpu.async_copy(x_buf.at[p], o_hbm.at[:, pl.ds(s0, _R), :],
                             sems_out[p]),
        ]

    def compute(c):
        p = c % 2

        def row(r, carry):
            def body(j, carry2):
                o16 = j * 16
                t = t_buf[p, r, pl.ds(o16, 16)]
                for b in range(_B):
                    x_buf[p, b, r, pl.ds(o16, 16)] = (
                        x_buf[p, b, r, pl.ds(o16, 16)] + t)
                return carry2

            return lax.fori_loop(0, _D // 16, body, carry, unroll=8)

        lax.fori_loop(0, _R, row, 0)

    start_in(0)
    for c in range(_C):
        p = c % 2
        if c + 1 < _C:
            if c >= 1:
                for d in copies_out[(c - 1) % 2]:
                    d.wait()
            start_in(c + 1)
        for d in copies_in[p]:
            d.wait()
        compute(c)
        start_out(c)
    for d in copies_out[_C % 2]:
        d.wait()
    for d in copies_out[(_C - 1) % 2]:
        d.wait()


def kernel(x, table):
    return _sc_add(x, table)
